# baseline (speedup 1.0000x reference)
import functools

import jax
import jax.numpy as jnp
from jax import lax
from jax.experimental import pallas as pl
from jax.experimental.pallas import tpu as pltpu
from jax.experimental.pallas import tpu_sc as plsc

NC = 2
NS = 16
NW = NC * NS
CHUNK = 128
HALF = 4


NBUF = 8


def _seg_sum_body(n_nodes, rows_per_tile, n0, n1,
                  h_hbm, src_hbm, dst_hbm, zeros_hbm, out_hbm,
                  src_v, dst_v, rows_v, acc, gsem):
    c = lax.axis_index("c")
    s = lax.axis_index("s")

    pltpu.sync_copy(zeros_hbm, acc.at[pl.ds(s * rows_per_tile, rows_per_tile)])

    @pl.when(c == 0)
    def _():
        pltpu.sync_copy(src_hbm.at[pl.ds(s * n0, n0)], src_v.at[pl.ds(0, n0)])
        pltpu.sync_copy(dst_hbm.at[pl.ds(s * n0, n0)], dst_v.at[pl.ds(0, n0)])

    @pl.when(c == 1)
    def _():
        base = NS * n0 + s * n1
        pltpu.sync_copy(src_hbm.at[pl.ds(base, n1)], src_v.at[pl.ds(0, n1)])
        pltpu.sync_copy(dst_hbm.at[pl.ds(base, n1)], dst_v.at[pl.ds(0, n1)])

    plsc.subcore_barrier()

    def body(j, carry):
        pltpu.async_copy(h_hbm.at[src_v.at[j]], rows_v, gsem).wait()
        pltpu.sync_copy(rows_v, acc.at[dst_v.at[j]], add=True)
        return carry

    lax.fori_loop(0, jnp.where(c == 0, n0, n1), body, 0)
    plsc.subcore_barrier()

    tail = n_nodes - (NS - 1) * rows_per_tile

    @pl.when(s < NS - 1)
    def _():
        pltpu.sync_copy(
            acc.at[pl.ds(s * rows_per_tile, rows_per_tile)],
            out_hbm.at[c, pl.ds(s * rows_per_tile, rows_per_tile)])

    @pl.when(s == NS - 1)
    def _():
        pltpu.sync_copy(
            acc.at[pl.ds((NS - 1) * rows_per_tile, tail)],
            out_hbm.at[c, pl.ds((NS - 1) * rows_per_tile, tail)])


def _segment_sum_sc(h, src3, dst3, zeros, n_nodes, acc_rows, n0, n1):
    d = h.shape[1]
    rows_per_tile = acc_rows // NS
    body = functools.partial(_seg_sum_body, n_nodes, rows_per_tile, n0, n1)
    kern = pl.kernel(
        body,
        mesh=plsc.VectorSubcoreMesh(core_axis_name="c", subcore_axis_name="s"),
        compiler_params=pltpu.CompilerParams(use_tc_tiling_on_sc=False),
        out_type=jax.ShapeDtypeStruct((NC, n_nodes, d), jnp.float32),
        scratch_types=[
            pltpu.VMEM((max(n0, n1), CHUNK), jnp.int32),
            pltpu.VMEM((max(n0, n1), CHUNK), jnp.int32),
            pltpu.VMEM((CHUNK, d), jnp.float32),
            pltpu.VMEM_SHARED((acc_rows, d), jnp.float32),
            pltpu.SemaphoreType.DMA,
        ],
    )
    return kern(h, src3, dst3, zeros)


def _tc1_body(x_ref, w1_ref, wr1_ref, br1_ref, h_ref, res_ref):
    x = x_ref[...]
    h_ref[...] = jnp.dot(x, w1_ref[...], preferred_element_type=jnp.float32)
    res = jnp.dot(x, wr1_ref[...], preferred_element_type=jnp.float32)
    res_ref[...] = jnp.maximum(res + br1_ref[...], 0.0)


def _bn(t, g, b):
    m = jnp.mean(t, axis=0, keepdims=True)
    v = jnp.mean(jnp.square(t - m), axis=0, keepdims=True)
    return (t - m) * jax.lax.rsqrt(v + 1e-5) * g + b


def _tc2_body(aggp_ref, b1_ref, res_ref, g1_ref, be1_ref, h1_ref):
    agg = aggp_ref[0] + aggp_ref[1]
    rst = jnp.maximum(agg + b1_ref[...], 0.0)
    h1_ref[...] = _bn(rst + res_ref[...], g1_ref[...], be1_ref[...])


def _tc3_body(agg2p_ref, w2_ref, b2_ref, h1_ref, wr2_ref, br2_ref,
              g2_ref, be2_ref, wg_ref, bg_ref, out_ref):
    agg2 = agg2p_ref[0] + agg2p_ref[1]
    rst2 = jnp.dot(agg2, w2_ref[...], preferred_element_type=jnp.float32)
    rst2 = jnp.maximum(rst2 + b2_ref[...], 0.0)
    res2 = jnp.dot(h1_ref[...], wr2_ref[...], preferred_element_type=jnp.float32)
    res2 = jnp.maximum(res2 + br2_ref[...], 0.0)
    h2 = _bn(rst2 + res2, g2_ref[...], be2_ref[...])
    wlog = jnp.dot(h2, wg_ref[...], preferred_element_type=jnp.float32)
    w = jax.nn.sigmoid(wlog + bg_ref[...])
    d = h2.shape[1]
    out_ref[:, :d] = jnp.sum(w * h2, axis=0, keepdims=True)
    out_ref[:, d:] = jnp.max(h2, axis=0, keepdims=True)


def kernel(x, edge_index, W1, b1, Wr1, br1, g1, be1, W2, b2, Wr2, br2,
           g2, be2, Wg, bg):
    n, d_in = x.shape
    d_h = W1.shape[1]
    e = edge_index.shape[1]

    cpw = -(-e // (NW * CHUNK))
    pair = 2 * cpw
    n0 = max(1, round(pair * 0.39))
    n1 = pair - n0
    ep = NS * pair * CHUNK
    acc_rows = -(-(n + 1) // (NS * 8)) * (NS * 8)
    src = edge_index[0]
    dst = edge_index[1]
    pad_dst = n + jnp.arange(ep - e, dtype=jnp.int32) % (acc_rows - n)
    src3 = jnp.concatenate(
        [src, jnp.zeros((ep - e,), jnp.int32)]).reshape(NS * pair, CHUNK)
    dst3 = jnp.concatenate([dst, pad_dst]).reshape(NS * pair, CHUNK)
    zeros = jnp.zeros((acc_rows // NS, d_h), jnp.float32)

    h, res = pl.pallas_call(
        _tc1_body,
        out_shape=[jax.ShapeDtypeStruct((n, d_h), jnp.float32),
                   jax.ShapeDtypeStruct((n, d_h), jnp.float32)],
    )(x, W1, Wr1, br1.reshape(1, d_h))

    aggp = _segment_sum_sc(h, src3, dst3, zeros, n, acc_rows, n0, n1)

    h1 = pl.pallas_call(
        _tc2_body,
        out_shape=jax.ShapeDtypeStruct((n, d_h), jnp.float32),
    )(aggp, b1.reshape(1, d_h), res, g1.reshape(1, d_h), be1.reshape(1, d_h))

    agg2p = _segment_sum_sc(h1, src3, dst3, zeros, n, acc_rows, n0, n1)

    out = pl.pallas_call(
        _tc3_body,
        out_shape=jax.ShapeDtypeStruct((1, 2 * d_h), jnp.float32),
    )(agg2p, W2, b2.reshape(1, d_h), h1, Wr2, br2.reshape(1, d_h),
      g2.reshape(1, d_h), be2.reshape(1, d_h), Wg, bg.reshape(1, 1))
    return out


# speedup vs baseline: 1.0859x; 1.0859x over previous
import functools

import jax
import jax.numpy as jnp
from jax import lax
from jax.experimental import pallas as pl
from jax.experimental.pallas import tpu as pltpu
from jax.experimental.pallas import tpu_sc as plsc

NC = 2
NS = 16
NW = NC * NS
CHUNK = 128


def _seg_sum_body(n_nodes, rows_per_tile, n_chunks,
                  h_hbm, src_hbm, dst_hbm, zeros_hbm, out_hbm,
                  src_v, dst_v, rows_v, acc, gsem):
    c = lax.axis_index("c")
    s = lax.axis_index("s")
    wid = s * NC + c

    pltpu.sync_copy(zeros_hbm, acc.at[pl.ds(s * rows_per_tile, rows_per_tile)])
    pltpu.sync_copy(src_hbm.at[wid], src_v)
    pltpu.sync_copy(dst_hbm.at[wid], dst_v)
    plsc.subcore_barrier()

    def body(j, carry):
        pltpu.async_copy(h_hbm.at[src_v.at[j]], rows_v, gsem).wait()
        pltpu.sync_copy(rows_v, acc.at[dst_v.at[j]], add=True)
        return carry

    lax.fori_loop(0, n_chunks, body, 0)
    plsc.subcore_barrier()

    tail = n_nodes - (NS - 1) * rows_per_tile

    @pl.when(s < NS - 1)
    def _():
        pltpu.sync_copy(
            acc.at[pl.ds(s * rows_per_tile, rows_per_tile)],
            out_hbm.at[c, pl.ds(s * rows_per_tile, rows_per_tile)])

    @pl.when(s == NS - 1)
    def _():
        pltpu.sync_copy(
            acc.at[pl.ds((NS - 1) * rows_per_tile, tail)],
            out_hbm.at[c, pl.ds((NS - 1) * rows_per_tile, tail)])


def _segment_sum_sc(h, src3, dst3, zeros, n_nodes, acc_rows):
    d = h.shape[1]
    n_chunks = src3.shape[1]
    rows_per_tile = acc_rows // NS
    body = functools.partial(_seg_sum_body, n_nodes, rows_per_tile, n_chunks)
    kern = pl.kernel(
        body,
        mesh=plsc.VectorSubcoreMesh(core_axis_name="c", subcore_axis_name="s"),
        compiler_params=pltpu.CompilerParams(use_tc_tiling_on_sc=False),
        out_type=jax.ShapeDtypeStruct((NC, n_nodes, d), jnp.float32),
        scratch_types=[
            pltpu.VMEM((n_chunks, CHUNK), jnp.int32),
            pltpu.VMEM((n_chunks, CHUNK), jnp.int32),
            pltpu.VMEM((CHUNK, d), jnp.float32),
            pltpu.VMEM_SHARED((acc_rows, d), jnp.float32),
            pltpu.SemaphoreType.DMA,
        ],
    )
    return kern(h, src3, dst3, zeros)


def _tc1_body(x_ref, w1_ref, wr1_ref, br1_ref, h_ref, res_ref):
    x = x_ref[...]
    h_ref[...] = jnp.dot(x, w1_ref[...], preferred_element_type=jnp.float32)
    res = jnp.dot(x, wr1_ref[...], preferred_element_type=jnp.float32)
    res_ref[...] = jnp.maximum(res + br1_ref[...], 0.0)


def _bn(t, g, b):
    m = jnp.mean(t, axis=0, keepdims=True)
    v = jnp.mean(jnp.square(t - m), axis=0, keepdims=True)
    return (t - m) * jax.lax.rsqrt(v + 1e-5) * g + b


def _tc2_body(aggp_ref, b1_ref, res_ref, g1_ref, be1_ref, h1_ref):
    agg = aggp_ref[0] + aggp_ref[1]
    rst = jnp.maximum(agg + b1_ref[...], 0.0)
    h1_ref[...] = _bn(rst + res_ref[...], g1_ref[...], be1_ref[...])


def _tc3_body(agg2p_ref, w2_ref, b2_ref, h1_ref, wr2_ref, br2_ref,
              g2_ref, be2_ref, wg_ref, bg_ref, out_ref):
    agg2 = agg2p_ref[0] + agg2p_ref[1]
    rst2 = jnp.dot(agg2, w2_ref[...], preferred_element_type=jnp.float32)
    rst2 = jnp.maximum(rst2 + b2_ref[...], 0.0)
    res2 = jnp.dot(h1_ref[...], wr2_ref[...], preferred_element_type=jnp.float32)
    res2 = jnp.maximum(res2 + br2_ref[...], 0.0)
    h2 = _bn(rst2 + res2, g2_ref[...], be2_ref[...])
    wlog = jnp.dot(h2, wg_ref[...], preferred_element_type=jnp.float32)
    w = jax.nn.sigmoid(wlog + bg_ref[...])
    d = h2.shape[1]
    out_ref[:, :d] = jnp.sum(w * h2, axis=0, keepdims=True)
    out_ref[:, d:] = jnp.max(h2, axis=0, keepdims=True)


def kernel(x, edge_index, W1, b1, Wr1, br1, g1, be1, W2, b2, Wr2, br2,
           g2, be2, Wg, bg):
    n, d_in = x.shape
    d_h = W1.shape[1]
    e = edge_index.shape[1]

    cpw = -(-e // (NW * CHUNK))
    ep = NW * cpw * CHUNK
    acc_rows = -(-(n + 1) // (NS * 8)) * (NS * 8)
    src = edge_index[0]
    dst = edge_index[1]
    pad_dst = n + jnp.arange(ep - e, dtype=jnp.int32) % (acc_rows - n)
    src3 = jnp.concatenate(
        [src, jnp.zeros((ep - e,), jnp.int32)]).reshape(NW, cpw, CHUNK)
    dst3 = jnp.concatenate([dst, pad_dst]).reshape(NW, cpw, CHUNK)
    zeros = jnp.zeros((acc_rows // NS, d_h), jnp.float32)

    h, res = pl.pallas_call(
        _tc1_body,
        out_shape=[jax.ShapeDtypeStruct((n, d_h), jnp.float32),
                   jax.ShapeDtypeStruct((n, d_h), jnp.float32)],
    )(x, W1, Wr1, br1.reshape(1, d_h))

    aggp = _segment_sum_sc(h, src3, dst3, zeros, n, acc_rows)

    h1 = pl.pallas_call(
        _tc2_body,
        out_shape=jax.ShapeDtypeStruct((n, d_h), jnp.float32),
    )(aggp, b1.reshape(1, d_h), res, g1.reshape(1, d_h), be1.reshape(1, d_h))

    agg2p = _segment_sum_sc(h1, src3, dst3, zeros, n, acc_rows)

    out = pl.pallas_call(
        _tc3_body,
        out_shape=jax.ShapeDtypeStruct((1, 2 * d_h), jnp.float32),
    )(agg2p, W2, b2.reshape(1, d_h), h1, Wr2, br2.reshape(1, d_h),
      g2.reshape(1, d_h), be2.reshape(1, d_h), Wg, bg.reshape(1, 1))
    return out
